# baseline (speedup 1.0000x reference)
import jax
import jax.numpy as jnp
from jax.experimental import pallas as pl
from jax.experimental.pallas import tpu as pltpu
from jax.experimental.pallas import tpu_sc as plsc

_SPB = 4


def _sc_l1_call(mlT, mpT, mtT, mel_len):
    B, NM, T = mlT.shape
    ml2 = mlT.reshape(B * NM, T)
    mp2 = mpT.reshape(B * NM, T)
    mt2 = mtT.reshape(B * NM, T)
    vector_mesh = plsc.VectorSubcoreMesh(core_axis_name="c", subcore_axis_name="s")
    lane_ids = jnp.arange(16, dtype=jnp.int32)
    len_b = jnp.broadcast_to(mel_len[:, None], (B, 16))

    @pl.kernel(out_type=jax.ShapeDtypeStruct((B, 2, 16), jnp.float32),
               mesh=vector_mesh,
               scratch_types=[pltpu.VMEM((2, 16), jnp.float32),
                              pltpu.VMEM((16,), jnp.int32),
                              pltpu.VMEM((16,), jnp.int32),
                              pltpu.SemaphoreType.DMA])
    def k(ml_hbm, mp_hbm, mt_hbm, len_hbm, lid_hbm, o_hbm,
          acc_vmem, lid_vmem, len_vmem, sem):
        cid = jax.lax.axis_index("c")
        sid = jax.lax.axis_index("s")
        tile = cid * 16 + sid
        pltpu.async_copy(len_hbm.at[tile], len_vmem, sem).wait()
        pltpu.async_copy(lid_hbm, lid_vmem, sem).wait()
        m_len = len_vmem[:]
        lanes = lid_vmem[:]
        acc_vmem[0, :] = jnp.zeros((16,), jnp.float32)
        acc_vmem[1, :] = jnp.zeros((16,), jnp.float32)

        def body(ml_v, mp_v, mt_v):
            @pl.loop(0, T, step=16)
            def _(c1):
                ok = (c1 + lanes) < m_len
                zero = jnp.zeros((16,), jnp.float32)
                al = acc_vmem[0, :]
                ap = acc_vmem[1, :]
                for r in range(8):
                    mt_r = mt_v[r, pl.ds(c1, 16)]
                    al += jnp.where(ok, jnp.abs(ml_v[r, pl.ds(c1, 16)] - mt_r), zero)
                    ap += jnp.where(ok, jnp.abs(mp_v[r, pl.ds(c1, 16)] - mt_r), zero)
                acc_vmem[0, :] = al
                acc_vmem[1, :] = ap

        row0 = tile * NM
        pltpu.emit_pipeline(
            body,
            grid=(NM // 8,),
            in_specs=[pl.BlockSpec((8, T), lambda i: (i, 0))] * 3,
            out_specs=[],
        )(ml_hbm.at[pl.ds(row0, NM)],
          mp_hbm.at[pl.ds(row0, NM)],
          mt_hbm.at[pl.ds(row0, NM)])

        pltpu.async_copy(acc_vmem, o_hbm.at[tile], sem).wait()

    return k(ml2, mp2, mt2, len_b, lane_ids)


def _tc_kernel(mel_len_ref, seq_len_ref, go_ref, gt_ref, al2_ref, al3_ref,
               out_ref, acc_ref):
    step = pl.program_id(0)
    nsteps = pl.num_programs(0)

    @pl.when(step == 0)
    def _init():
        for i in range(4):
            acc_ref[i] = 0.0

    s_bce = 0.0
    s_guide = 0.0
    n_sel = 0.0
    den_w = 0.0
    for j in range(_SPB):
        b = step * _SPB + j
        m_len = mel_len_ref[b]
        s_len = seq_len_ref[b]
        m_len_f = m_len.astype(jnp.float32)
        s_len_f = s_len.astype(jnp.float32)

        x = go_ref[pl.ds(b, 1), :]
        z = gt_ref[pl.ds(b, 1), :]
        t_idx = jax.lax.broadcasted_iota(jnp.int32, x.shape, 1)
        gmask = t_idx < m_len
        bce = jnp.maximum(x, 0.0) - x * z + jnp.log1p(jnp.exp(-jnp.abs(x)))
        s_bce += jnp.sum(jnp.where(gmask, bce, 0.0))

        a = al2_ref[j, 0] + al3_ref[j, 0]
        ll_i = jax.lax.broadcasted_iota(jnp.int32, a.shape, 0) + 1
        tt_i = jax.lax.broadcasted_iota(jnp.int32, a.shape, 1) + 1
        tt = tt_i.astype(jnp.float32)
        ll = ll_i.astype(jnp.float32)
        diff = tt * (1.0 / m_len_f) - ll * (1.0 / s_len_f)
        w = 1.0 - jnp.exp(-1.25 * diff * diff)
        inside = (tt_i <= m_len) & (ll_i <= s_len)
        s_guide += jnp.sum(jnp.where(inside, a * w, 0.0))

        n_sel += m_len_f
        den_w += m_len_f * s_len_f

    acc_ref[0] += s_bce
    acc_ref[1] += s_guide
    acc_ref[2] += n_sel
    acc_ref[3] += den_w

    @pl.when(step == nsteps - 1)
    def _finish():
        for i in range(4):
            out_ref[i] = acc_ref[i]


def kernel(mel_linear, mel_post, gate_out, mel_target, gate_target, mel_mask, mel_len, seq_len, alignments):
    B, T, NM = mel_linear.shape
    _, H, _, L = alignments.shape

    mlT = jnp.transpose(mel_linear, (0, 2, 1))
    mpT = jnp.transpose(mel_post, (0, 2, 1))
    mtT = jnp.transpose(mel_target, (0, 2, 1))
    alT = jnp.transpose(alignments, (0, 1, 3, 2))

    sc_partials = _sc_l1_call(mlT, mpT, mtT, mel_len)

    scalar_spec = pl.BlockSpec(memory_space=pltpu.SMEM)
    in_specs = [
        scalar_spec,
        scalar_spec,
        pl.BlockSpec((B, T), lambda i: (0, 0)),
        pl.BlockSpec((B, T), lambda i: (0, 0)),
        pl.BlockSpec((_SPB, 1, L, T), lambda i: (i, 2, 0, 0)),
        pl.BlockSpec((_SPB, 1, L, T), lambda i: (i, 3, 0, 0)),
    ]
    tc_out = pl.pallas_call(
        _tc_kernel,
        grid=(B // _SPB,),
        in_specs=in_specs,
        out_specs=pl.BlockSpec(memory_space=pltpu.SMEM),
        out_shape=jax.ShapeDtypeStruct((4,), jnp.float32),
        scratch_shapes=[pltpu.SMEM((4,), jnp.float32)],
    )(mel_len, seq_len, gate_out, gate_target, alT, alT)

    s_bce, s_guide, n_sel, den_w = tc_out[0], tc_out[1], tc_out[2], tc_out[3]
    s_lin = jnp.sum(sc_partials[:, 0, :])
    s_post = jnp.sum(sc_partials[:, 1, :])

    mel_linear_loss = s_lin / (n_sel * NM)
    mel_post_loss = s_post / (n_sel * NM)
    gate_loss = s_bce / n_sel
    guide_loss = s_guide / (2.0 * den_w)
    return (mel_linear_loss, mel_post_loss, gate_loss, guide_loss)


# speedup vs baseline: 1.0011x; 1.0011x over previous
import jax
import jax.numpy as jnp
from jax.experimental import pallas as pl
from jax.experimental.pallas import tpu as pltpu
from jax.experimental.pallas import tpu_sc as plsc

_SPB = 4


def _sc_l1_call(mlT, mpT, mtT, mel_len):
    B, NM, T = mlT.shape
    ml2 = mlT.reshape(B * NM, T)
    mp2 = mpT.reshape(B * NM, T)
    mt2 = mtT.reshape(B * NM, T)
    vector_mesh = plsc.VectorSubcoreMesh(core_axis_name="c", subcore_axis_name="s")
    lane_ids = jnp.arange(16, dtype=jnp.int32)
    len_b = jnp.broadcast_to(mel_len[:, None], (B, 16))

    @pl.kernel(out_type=jax.ShapeDtypeStruct((B, 2, 16), jnp.float32),
               mesh=vector_mesh,
               scratch_types=[pltpu.VMEM((2, 16), jnp.float32),
                              pltpu.VMEM((16,), jnp.int32),
                              pltpu.VMEM((16,), jnp.int32),
                              pltpu.SemaphoreType.DMA])
    def k(ml_hbm, mp_hbm, mt_hbm, len_hbm, lid_hbm, o_hbm,
          acc_vmem, lid_vmem, len_vmem, sem):
        cid = jax.lax.axis_index("c")
        sid = jax.lax.axis_index("s")
        tile = cid * 16 + sid
        pltpu.async_copy(len_hbm.at[tile], len_vmem, sem).wait()
        pltpu.async_copy(lid_hbm, lid_vmem, sem).wait()
        m_len = len_vmem[:]
        lanes = lid_vmem[:]
        acc_vmem[0, :] = jnp.zeros((16,), jnp.float32)
        acc_vmem[1, :] = jnp.zeros((16,), jnp.float32)

        def body(ml_v, mp_v, mt_v):
            def chunk(i, carry):
                al, ap = carry
                c1 = i * 16
                ok = (c1 + lanes) < m_len
                zero = jnp.zeros((16,), jnp.float32)
                for r in range(8):
                    mt_r = mt_v[r, pl.ds(c1, 16)]
                    al += jnp.where(ok, jnp.abs(ml_v[r, pl.ds(c1, 16)] - mt_r), zero)
                    ap += jnp.where(ok, jnp.abs(mp_v[r, pl.ds(c1, 16)] - mt_r), zero)
                return al, ap
            al, ap = jax.lax.fori_loop(
                0, T // 16, chunk, (acc_vmem[0, :], acc_vmem[1, :]))
            acc_vmem[0, :] = al
            acc_vmem[1, :] = ap

        row0 = tile * NM
        pltpu.emit_pipeline(
            body,
            grid=(NM // 8,),
            in_specs=[pl.BlockSpec((8, T), lambda i: (i, 0))] * 3,
            out_specs=[],
        )(ml_hbm.at[pl.ds(row0, NM)],
          mp_hbm.at[pl.ds(row0, NM)],
          mt_hbm.at[pl.ds(row0, NM)])

        pltpu.async_copy(acc_vmem, o_hbm.at[tile], sem).wait()

    return k(ml2, mp2, mt2, len_b, lane_ids)


def _tc_kernel(mel_len_ref, seq_len_ref, go_ref, gt_ref, al2_ref, al3_ref,
               out_ref, acc_ref):
    step = pl.program_id(0)
    nsteps = pl.num_programs(0)

    @pl.when(step == 0)
    def _init():
        for i in range(4):
            acc_ref[i] = 0.0

    s_bce = 0.0
    s_guide = 0.0
    n_sel = 0.0
    den_w = 0.0
    for j in range(_SPB):
        b = step * _SPB + j
        m_len = mel_len_ref[b]
        s_len = seq_len_ref[b]
        m_len_f = m_len.astype(jnp.float32)
        s_len_f = s_len.astype(jnp.float32)

        x = go_ref[pl.ds(b, 1), :]
        z = gt_ref[pl.ds(b, 1), :]
        t_idx = jax.lax.broadcasted_iota(jnp.int32, x.shape, 1)
        gmask = t_idx < m_len
        bce = jnp.maximum(x, 0.0) - x * z + jnp.log1p(jnp.exp(-jnp.abs(x)))
        s_bce += jnp.sum(jnp.where(gmask, bce, 0.0))

        a = al2_ref[j, 0] + al3_ref[j, 0]
        ll_i = jax.lax.broadcasted_iota(jnp.int32, a.shape, 0) + 1
        tt_i = jax.lax.broadcasted_iota(jnp.int32, a.shape, 1) + 1
        tt = tt_i.astype(jnp.float32)
        ll = ll_i.astype(jnp.float32)
        diff = tt * (1.0 / m_len_f) - ll * (1.0 / s_len_f)
        w = 1.0 - jnp.exp(-1.25 * diff * diff)
        inside = (tt_i <= m_len) & (ll_i <= s_len)
        s_guide += jnp.sum(jnp.where(inside, a * w, 0.0))

        n_sel += m_len_f
        den_w += m_len_f * s_len_f

    acc_ref[0] += s_bce
    acc_ref[1] += s_guide
    acc_ref[2] += n_sel
    acc_ref[3] += den_w

    @pl.when(step == nsteps - 1)
    def _finish():
        for i in range(4):
            out_ref[i] = acc_ref[i]


def kernel(mel_linear, mel_post, gate_out, mel_target, gate_target, mel_mask, mel_len, seq_len, alignments):
    B, T, NM = mel_linear.shape
    _, H, _, L = alignments.shape

    mlT = jnp.transpose(mel_linear, (0, 2, 1))
    mpT = jnp.transpose(mel_post, (0, 2, 1))
    mtT = jnp.transpose(mel_target, (0, 2, 1))
    alT = jnp.transpose(alignments, (0, 1, 3, 2))

    sc_partials = _sc_l1_call(mlT, mpT, mtT, mel_len)

    scalar_spec = pl.BlockSpec(memory_space=pltpu.SMEM)
    in_specs = [
        scalar_spec,
        scalar_spec,
        pl.BlockSpec((B, T), lambda i: (0, 0)),
        pl.BlockSpec((B, T), lambda i: (0, 0)),
        pl.BlockSpec((_SPB, 1, L, T), lambda i: (i, 2, 0, 0)),
        pl.BlockSpec((_SPB, 1, L, T), lambda i: (i, 3, 0, 0)),
    ]
    tc_out = pl.pallas_call(
        _tc_kernel,
        grid=(B // _SPB,),
        in_specs=in_specs,
        out_specs=pl.BlockSpec(memory_space=pltpu.SMEM),
        out_shape=jax.ShapeDtypeStruct((4,), jnp.float32),
        scratch_shapes=[pltpu.SMEM((4,), jnp.float32)],
    )(mel_len, seq_len, gate_out, gate_target, alT, alT)

    s_bce, s_guide, n_sel, den_w = tc_out[0], tc_out[1], tc_out[2], tc_out[3]
    s_lin = jnp.sum(sc_partials[:, 0, :])
    s_post = jnp.sum(sc_partials[:, 1, :])

    mel_linear_loss = s_lin / (n_sel * NM)
    mel_post_loss = s_post / (n_sel * NM)
    gate_loss = s_bce / n_sel
    guide_loss = s_guide / (2.0 * den_w)
    return (mel_linear_loss, mel_post_loss, gate_loss, guide_loss)
